# Initial kernel scaffold; baseline (speedup 1.0000x reference)
import jax
import jax.numpy as jnp
from jax.experimental import pallas as pl


def kernel(x, edge_index, W1, b1, W2, b2, W3, b3, W4, b4, W5, b5, W6, b6, Wc, bc):
    raise NotImplementedError("write your pallas kernel here")


# speedup vs baseline: 8.7063x; 8.7063x over previous
import functools

import jax
import jax.numpy as jnp
from jax import lax
from jax.experimental import pallas as pl
from jax.experimental.pallas import tpu as pltpu
from jax.experimental.pallas import tpu_sc as plsc

N = 10000
E = 320000
NC = 2
NS = 16
CHUNK = 128
ACC_ROWS = 10240
ROWS_PER_TILE_Z = ACC_ROWS // NS
ROWS_PER_TILE_W = N // NS

K_L = -(-E // (NS * CHUNK))
E_L = NS * K_L * CHUNK
K_H = -(-E // (NC * NS * CHUNK))
E_H = NC * NS * K_H * CHUNK

_BLOCKING = {512: (8, 64), 256: (4, 64), 128: (2, 64), 64: (2, 32),
             32: (2, 16), 16: (1, 16)}

_MESH = plsc.VectorSubcoreMesh(core_axis_name="c", subcore_axis_name="s",
                               num_cores=NC, num_subcores=NS)


def _fill(ref, rows, cb, value):
    def body(r, _):
        for cc in range(cb // 16):
            ref[r, pl.ds(16 * cc, 16)] = jnp.full((16,), value, jnp.float32)
        return 0
    lax.fori_loop(0, rows, body, 0)


def _make_hist():
    cb = 16

    def body(dst_hbm, out, dst_v, ones_v, zeros_v, acc):
        c = lax.axis_index("c")
        s = lax.axis_index("s")
        wid = c * NS + s
        pltpu.sync_copy(dst_hbm.at[wid], dst_v)
        _fill(ones_v, CHUNK, cb, 1.0)
        _fill(zeros_v, CHUNK, cb, 0.0)
        for m in range(ROWS_PER_TILE_Z // CHUNK):
            pltpu.sync_copy(zeros_v,
                            acc.at[pl.ds(s * ROWS_PER_TILE_Z + m * CHUNK, CHUNK)])
        plsc.subcore_barrier()

        def chunk(k, _):
            pltpu.sync_copy(ones_v, acc.at[dst_v.at[k]], add=True)
            return 0
        lax.fori_loop(0, K_H, chunk, 0)
        plsc.subcore_barrier()
        pltpu.sync_copy(acc.at[pl.ds(s * ROWS_PER_TILE_Z, ROWS_PER_TILE_Z)],
                        out.at[c, pl.ds(s * ROWS_PER_TILE_Z, ROWS_PER_TILE_Z)])

    return pl.kernel(
        body,
        out_type=jax.ShapeDtypeStruct((NC, ACC_ROWS, cb), jnp.float32),
        mesh=_MESH,
        scratch_types=[
            pltpu.VMEM((K_H, CHUNK), jnp.int32),
            pltpu.VMEM((CHUNK, cb), jnp.float32),
            pltpu.VMEM((CHUNK, cb), jnp.float32),
            pltpu.VMEM_SHARED((ACC_ROWS, cb), jnp.float32),
        ],
        compiler_params=pltpu.CompilerParams(use_tc_tiling_on_sc=False),
    )


def _make_scatter(f):
    nb, cb = _BLOCKING[f]

    def body(src_hbm, dst_hbm, *rest):
        xws_refs = rest[:nb]
        out = rest[nb]
        src_v, dst_v, rows_v, zeros_v, acc, sem = rest[nb + 1:]
        c = lax.axis_index("c")
        s = lax.axis_index("s")
        pltpu.sync_copy(src_hbm.at[s], src_v)
        pltpu.sync_copy(dst_hbm.at[s], dst_v)
        _fill(zeros_v, CHUNK, cb, 0.0)

        for core_id in range(NC):
            blocks = [j for j in range(nb) if j % NC == core_id]
            if not blocks:
                continue

            @pl.when(c == core_id)
            def _(blocks=blocks):
                for j in blocks:
                    for m in range(ROWS_PER_TILE_Z // CHUNK):
                        pltpu.sync_copy(
                            zeros_v,
                            acc.at[pl.ds(s * ROWS_PER_TILE_Z + m * CHUNK, CHUNK)])
                    plsc.subcore_barrier()

                    def chunk(k, _, j=j):
                        pltpu.async_copy(xws_refs[j].at[src_v.at[k]], rows_v,
                                         sem).wait()
                        pltpu.sync_copy(rows_v, acc.at[dst_v.at[k]], add=True)
                        return 0
                    lax.fori_loop(0, K_L, chunk, 0)
                    plsc.subcore_barrier()
                    pltpu.sync_copy(
                        acc.at[pl.ds(s * ROWS_PER_TILE_Z, ROWS_PER_TILE_Z)],
                        out.at[j, pl.ds(s * ROWS_PER_TILE_Z, ROWS_PER_TILE_Z)])
                    plsc.subcore_barrier()

    return pl.kernel(
        body,
        out_type=jax.ShapeDtypeStruct((nb, ACC_ROWS, cb), jnp.float32),
        mesh=_MESH,
        scratch_types=[
            pltpu.VMEM((K_L, CHUNK), jnp.int32),
            pltpu.VMEM((K_L, CHUNK), jnp.int32),
            pltpu.VMEM((CHUNK, cb), jnp.float32),
            pltpu.VMEM((CHUNK, cb), jnp.float32),
            pltpu.VMEM_SHARED((ACC_ROWS, cb), jnp.float32),
            pltpu.SemaphoreType.DMA,
        ],
        compiler_params=pltpu.CompilerParams(use_tc_tiling_on_sc=False),
    )


ROW_TILE = 400


def _pre_body(hist_ref, x_ref, w_ref, dis_ref, *out_refs):
    deg = hist_ref[0, :, 0] + hist_ref[1, :, 0] + 1.0
    dis = lax.rsqrt(deg)
    dis_ref[...] = dis[:, None]
    xws = jnp.dot(x_ref[...], w_ref[...],
                  preferred_element_type=jnp.float32) * dis[:, None]
    cb = out_refs[0].shape[1]
    for j, o in enumerate(out_refs):
        o[...] = xws[:, j * cb:(j + 1) * cb]


def _layer_body(nbp, relu, s_ref, dis_ref, b_ref, w_ref, *rest):
    prev_refs = rest[:nbp]
    out_refs = rest[nbp:]
    xwsp = jnp.concatenate([r[...] for r in prev_refs], axis=1)
    s_full = s_ref[...]
    s_tile = jnp.concatenate([s_full[j] for j in range(s_full.shape[0])],
                             axis=1)
    dis = dis_ref[...]
    h = dis * (s_tile + xwsp) + b_ref[...]
    if relu:
        h = jnp.maximum(h, 0.0)
    y = jnp.dot(h, w_ref[...], preferred_element_type=jnp.float32) * dis
    cb = out_refs[0].shape[1]
    for j, o in enumerate(out_refs):
        o[...] = y[:, j * cb:(j + 1) * cb]


def _tc_pre(hist, x, w1):
    nb, cb = _BLOCKING[w1.shape[1]]
    grid = N // ROW_TILE
    return pl.pallas_call(
        _pre_body,
        grid=(grid,),
        in_specs=[
            pl.BlockSpec((NC, ROW_TILE, 16), lambda i: (0, i, 0)),
            pl.BlockSpec((ROW_TILE, x.shape[1]), lambda i: (i, 0)),
            pl.BlockSpec(w1.shape, lambda i: (0, 0)),
        ],
        out_specs=[pl.BlockSpec((ROW_TILE, 1), lambda i: (i, 0))] +
                  [pl.BlockSpec((ROW_TILE, cb), lambda i: (i, 0))] * nb,
        out_shape=[jax.ShapeDtypeStruct((N, 1), jnp.float32)] +
                  [jax.ShapeDtypeStruct((N, cb), jnp.float32)] * nb,
    )(hist, x, w1)


def _tc_layer(s, dis, b, w, xws_blocks, relu=True):
    f_in = w.shape[0]
    nbp = len(xws_blocks)
    cbp = xws_blocks[0].shape[1]
    nbs, cbs = s.shape[0], s.shape[2]
    nb, cb = _BLOCKING[w.shape[1]]
    grid = N // ROW_TILE
    return pl.pallas_call(
        functools.partial(_layer_body, nbp, relu),
        grid=(grid,),
        in_specs=[
            pl.BlockSpec((nbs, ROW_TILE, cbs), lambda i: (0, i, 0)),
            pl.BlockSpec((ROW_TILE, 1), lambda i: (i, 0)),
            pl.BlockSpec((1, f_in), lambda i: (0, 0)),
            pl.BlockSpec(w.shape, lambda i: (0, 0)),
        ] + [pl.BlockSpec((ROW_TILE, cbp), lambda i: (i, 0))] * nbp,
        out_specs=[pl.BlockSpec((ROW_TILE, cb), lambda i: (i, 0))] * nb,
        out_shape=[jax.ShapeDtypeStruct((N, cb), jnp.float32)] * nb,
    )(s, dis, b.reshape(1, f_in), w, *xws_blocks)


def _final_body(s_ref, xws_ref, dis_ref, b_ref, wc_ref, bc_ref, out_ref):
    s6 = s_ref[...][0, :N, :]
    h6 = dis_ref[...] * (s6 + xws_ref[...]) + b_ref[...]
    m = jnp.mean(h6, axis=0, keepdims=True)
    out_ref[...] = jnp.dot(m, wc_ref[...],
                           preferred_element_type=jnp.float32) + bc_ref[...]


def _tc_final(s6, xws6, dis, b6, wc, bc):
    return pl.pallas_call(
        _final_body,
        out_shape=jax.ShapeDtypeStruct((1, wc.shape[1]), jnp.float32),
    )(s6, xws6, dis, b6.reshape(1, -1), wc, bc.reshape(1, -1))


def kernel(x, edge_index, W1, b1, W2, b2, W3, b3, W4, b4, W5, b5, W6, b6,
           Wc, bc):
    src = edge_index[0].astype(jnp.int32)
    dst = edge_index[1].astype(jnp.int32)

    pad_h = E_H - E
    dst_h = jnp.concatenate([dst, jnp.full((pad_h,), N, jnp.int32)])
    dst_h = dst_h.reshape(NC * NS, K_H, CHUNK)
    pad_l = E_L - E
    src_l = jnp.concatenate([src, jnp.zeros((pad_l,), jnp.int32)])
    src_l = src_l.reshape(NS, K_L, CHUNK)
    dst_l = jnp.concatenate([dst, jnp.full((pad_l,), N, jnp.int32)])
    dst_l = dst_l.reshape(NS, K_L, CHUNK)

    hist = _make_hist()(dst_h)

    pre = _tc_pre(hist, x, W1)
    dis, xws = pre[0], list(pre[1:])

    scat = {f: _make_scatter(f) for f in (512, 256, 128, 64, 32, 16)}

    s = scat[512](src_l, dst_l, *xws)
    xws = _tc_layer(s, dis, b1, W2, xws)
    s = scat[256](src_l, dst_l, *xws)
    xws = _tc_layer(s, dis, b2, W3, xws)
    s = scat[128](src_l, dst_l, *xws)
    xws = _tc_layer(s, dis, b3, W4, xws)
    s = scat[64](src_l, dst_l, *xws)
    xws = _tc_layer(s, dis, b4, W5, xws)
    s = scat[32](src_l, dst_l, *xws)
    xws = _tc_layer(s, dis, b5, W6, xws)
    s = scat[16](src_l, dst_l, *xws)
    return _tc_final(s, xws[0], dis, b6, Wc, bc)
